# baseline (speedup 1.0000x reference)
import jax
import jax.numpy as jnp
import numpy as np
from jax import lax
from jax.experimental import pallas as pl
from jax.experimental.pallas import tpu as pltpu
from jax.experimental.pallas import tpu_sc as plsc

VOCAB = 100000
D_MODEL = 768
MAX_LEN = 8192
BATCH = 4
SEQ = 2048

NUM_CORES = 2
NUM_SUBCORES = 16
NUM_WORKERS = NUM_CORES * NUM_SUBCORES
S_PER_W = SEQ // NUM_WORKERS
HALF = S_PER_W // 2
STEPS = BATCH * 2
LANES = 16
GROUPS = D_MODEL // LANES


def _pos_encoding_np(max_len, d_model):
    pos = np.arange(max_len, dtype=np.float32)[:, None]
    i = np.arange(d_model, dtype=np.float32)[None, :]
    angle_rates = 1.0 / np.power(10000.0, (2.0 * np.floor(i / 2.0)) / d_model)
    angles = pos * angle_rates
    pe = np.zeros((max_len, d_model), dtype=np.float32)
    pe[:, 0::2] = np.sin(angles[:, 0::2])
    pe[:, 1::2] = np.cos(angles[:, 1::2])
    return pe


_PE = _pos_encoding_np(SEQ, D_MODEL)


def _emb_kernel(x_hbm, table_hbm, pe_hbm, out_hbm,
                idx_v, rows0, rows1, pe_v, g0, g1, w0, w1):
    wid = lax.axis_index("s") * NUM_CORES + lax.axis_index("c")
    s0 = wid * S_PER_W

    for b in range(BATCH):
        pltpu.sync_copy(x_hbm.at[b, pl.ds(s0, S_PER_W)], idx_v.at[b])

    bufs = (rows0, rows1)
    gsems = (g0, g1)
    wsems = (w0, w1)
    gathers = [None, None]
    writes = [None, None]

    def start_gather(k):
        b, h = divmod(k, 2)
        idx = idx_v.at[b, pl.ds(h * HALF, HALF)]
        gathers[k % 2] = pltpu.async_copy(
            table_hbm.at[idx], bufs[k % 2], gsems[k % 2])

    start_gather(0)

    pltpu.sync_copy(pe_hbm.at[pl.ds(s0, S_PER_W), :], pe_v)

    for k in range(STEPS):
        b, h = divmod(k, 2)
        buf = bufs[k % 2]
        gathers[k % 2].wait()

        def add_row(t, _, buf=buf, h=h):
            for g in range(GROUPS):
                sl = pl.ds(g * LANES, LANES)
                buf[t, sl] = buf[t, sl] + pe_v[h * HALF + t, sl]
            return _

        lax.fori_loop(0, HALF, add_row, 0)

        if k + 1 < STEPS:
            nxt = (k + 1) % 2
            if writes[nxt] is not None:
                writes[nxt].wait()
                writes[nxt] = None
            start_gather(k + 1)

        writes[k % 2] = pltpu.async_copy(
            buf, out_hbm.at[b, pl.ds(s0 + h * HALF, HALF), :], wsems[k % 2])

    for w in writes:
        if w is not None:
            w.wait()


@jax.jit
def kernel(x, tok_table):
    mesh = plsc.VectorSubcoreMesh(core_axis_name="c", subcore_axis_name="s")
    call = pl.kernel(
        _emb_kernel,
        out_type=jax.ShapeDtypeStruct((BATCH, SEQ, D_MODEL), jnp.float32),
        mesh=mesh,
        scratch_types=[
            pltpu.VMEM((BATCH, S_PER_W), jnp.int32),
            pltpu.VMEM((HALF, D_MODEL), jnp.float32),
            pltpu.VMEM((HALF, D_MODEL), jnp.float32),
            pltpu.VMEM((S_PER_W, D_MODEL), jnp.float32),
            pltpu.SemaphoreType.DMA,
            pltpu.SemaphoreType.DMA,
            pltpu.SemaphoreType.DMA,
            pltpu.SemaphoreType.DMA,
        ],
    )
    return call(x, tok_table, jnp.asarray(_PE))


# speedup vs baseline: 1.1331x; 1.1331x over previous
import jax
import jax.numpy as jnp
import numpy as np
from jax import lax
from jax.experimental import pallas as pl
from jax.experimental.pallas import tpu as pltpu
from jax.experimental.pallas import tpu_sc as plsc

VOCAB = 100000
D_MODEL = 768
MAX_LEN = 8192
BATCH = 4
SEQ = 2048

NUM_CORES = 2
NUM_SUBCORES = 16
NUM_WORKERS = NUM_CORES * NUM_SUBCORES
S_PER_W = SEQ // NUM_WORKERS
HALF = S_PER_W // 2
STEPS = BATCH * 2
LANES = 16
GROUPS = D_MODEL // LANES


def _pos_encoding_np(max_len, d_model):
    pos = np.arange(max_len, dtype=np.float32)[:, None]
    i = np.arange(d_model, dtype=np.float32)[None, :]
    angle_rates = 1.0 / np.power(10000.0, (2.0 * np.floor(i / 2.0)) / d_model)
    angles = pos * angle_rates
    pe = np.zeros((max_len, d_model), dtype=np.float32)
    pe[:, 0::2] = np.sin(angles[:, 0::2])
    pe[:, 1::2] = np.cos(angles[:, 1::2])
    return pe


_PE = _pos_encoding_np(SEQ, D_MODEL)


def _emb_kernel(x_hbm, table_hbm, pe_hbm, out_hbm,
                idx_v, rows0, rows1, pe_v, g0, g1, w0, w1):
    wid = lax.axis_index("s") * NUM_CORES + lax.axis_index("c")
    s0 = wid * S_PER_W

    for b in range(BATCH):
        pltpu.sync_copy(x_hbm.at[b, pl.ds(s0, S_PER_W)], idx_v.at[b])

    bufs = (rows0, rows1)
    gsems = (g0, g1)
    wsems = (w0, w1)
    gathers = [None, None]
    writes = [None, None]

    def start_gather(k):
        b, h = divmod(k, 2)
        idx = idx_v.at[b, pl.ds(h * HALF, HALF)]
        gathers[k % 2] = pltpu.async_copy(
            table_hbm.at[idx], bufs[k % 2], gsems[k % 2])

    start_gather(0)

    pltpu.sync_copy(pe_hbm.at[pl.ds(s0, S_PER_W), :], pe_v)

    for k in range(STEPS):
        b, h = divmod(k, 2)
        buf = bufs[k % 2]
        gathers[k % 2].wait()

        if k + 1 < STEPS:
            nxt = (k + 1) % 2
            if writes[nxt] is not None:
                writes[nxt].wait()
                writes[nxt] = None
            start_gather(k + 1)

        def add_row(t, _, buf=buf, h=h):
            for g in range(GROUPS):
                sl = pl.ds(g * LANES, LANES)
                buf[t, sl] = buf[t, sl] + pe_v[h * HALF + t, sl]
            return _

        lax.fori_loop(0, HALF, add_row, 0)

        writes[k % 2] = pltpu.async_copy(
            buf, out_hbm.at[b, pl.ds(s0 + h * HALF, HALF), :], wsems[k % 2])

    for w in writes:
        if w is not None:
            w.wait()


@jax.jit
def kernel(x, tok_table):
    mesh = plsc.VectorSubcoreMesh(core_axis_name="c", subcore_axis_name="s")
    call = pl.kernel(
        _emb_kernel,
        out_type=jax.ShapeDtypeStruct((BATCH, SEQ, D_MODEL), jnp.float32),
        mesh=mesh,
        scratch_types=[
            pltpu.VMEM((BATCH, S_PER_W), jnp.int32),
            pltpu.VMEM((HALF, D_MODEL), jnp.float32),
            pltpu.VMEM((HALF, D_MODEL), jnp.float32),
            pltpu.VMEM((S_PER_W, D_MODEL), jnp.float32),
            pltpu.SemaphoreType.DMA,
            pltpu.SemaphoreType.DMA,
            pltpu.SemaphoreType.DMA,
            pltpu.SemaphoreType.DMA,
        ],
    )
    return call(x, tok_table, jnp.asarray(_PE))


# speedup vs baseline: 1.2460x; 1.0997x over previous
import jax
import jax.numpy as jnp
import numpy as np
from jax import lax
from jax.experimental import pallas as pl
from jax.experimental.pallas import tpu as pltpu
from jax.experimental.pallas import tpu_sc as plsc

VOCAB = 100000
D_MODEL = 768
MAX_LEN = 8192
BATCH = 4
SEQ = 2048

NUM_CORES = 2
NUM_SUBCORES = 16
NUM_WORKERS = NUM_CORES * NUM_SUBCORES
S_PER_W = SEQ // NUM_WORKERS
HALF = S_PER_W // 2
STEPS = BATCH * 2
LANES = 16
GROUPS = D_MODEL // LANES


def _pos_encoding_np(max_len, d_model):
    pos = np.arange(max_len, dtype=np.float32)[:, None]
    i = np.arange(d_model, dtype=np.float32)[None, :]
    angle_rates = 1.0 / np.power(10000.0, (2.0 * np.floor(i / 2.0)) / d_model)
    angles = pos * angle_rates
    pe = np.zeros((max_len, d_model), dtype=np.float32)
    pe[:, 0::2] = np.sin(angles[:, 0::2])
    pe[:, 1::2] = np.cos(angles[:, 1::2])
    return pe


_PE = _pos_encoding_np(SEQ, D_MODEL)


def _emb_kernel(x_hbm, table_hbm, pe_hbm, out_hbm,
                idx_v, rows0, rows1, pe_v, g0, g1, w0, w1):
    wid = lax.axis_index("s") * NUM_CORES + lax.axis_index("c")
    s0 = wid * S_PER_W

    for b in range(BATCH):
        pltpu.sync_copy(x_hbm.at[b, pl.ds(s0, S_PER_W)], idx_v.at[b])

    bufs = (rows0, rows1)
    gsems = (g0, g1)
    wsems = (w0, w1)
    gathers = [None, None]
    writes = [None, None]

    def start_gather(k):
        b, h = divmod(k, 2)
        idx = idx_v.at[b, pl.ds(h * HALF, HALF)]
        gathers[k % 2] = pltpu.async_copy(
            table_hbm.at[idx], bufs[k % 2], gsems[k % 2])

    start_gather(0)

    pltpu.sync_copy(pe_hbm.at[pl.ds(s0, S_PER_W), :], pe_v)

    for k in range(STEPS):
        b, h = divmod(k, 2)
        buf = bufs[k % 2]
        gathers[k % 2].wait()

        if k + 1 < STEPS:
            nxt = (k + 1) % 2
            if writes[nxt] is not None:
                writes[nxt].wait()
                writes[nxt] = None
            start_gather(k + 1)

        def add_row(t, _, buf=buf, h=h):
            for g in range(GROUPS):
                sl = pl.ds(g * LANES, LANES)
                plsc.addupdate(buf.at[t, sl], pe_v[h * HALF + t, sl])
            return _

        lax.fori_loop(0, HALF, add_row, 0)

        writes[k % 2] = pltpu.async_copy(
            buf, out_hbm.at[b, pl.ds(s0 + h * HALF, HALF), :], wsems[k % 2])

    for w in writes:
        if w is not None:
            w.wait()


@jax.jit
def kernel(x, tok_table):
    mesh = plsc.VectorSubcoreMesh(core_axis_name="c", subcore_axis_name="s")
    call = pl.kernel(
        _emb_kernel,
        out_type=jax.ShapeDtypeStruct((BATCH, SEQ, D_MODEL), jnp.float32),
        mesh=mesh,
        scratch_types=[
            pltpu.VMEM((BATCH, S_PER_W), jnp.int32),
            pltpu.VMEM((HALF, D_MODEL), jnp.float32),
            pltpu.VMEM((HALF, D_MODEL), jnp.float32),
            pltpu.VMEM((S_PER_W, D_MODEL), jnp.float32),
            pltpu.SemaphoreType.DMA,
            pltpu.SemaphoreType.DMA,
            pltpu.SemaphoreType.DMA,
            pltpu.SemaphoreType.DMA,
        ],
    )
    return call(x, tok_table, jnp.asarray(_PE))
